# baseline (speedup 1.0000x reference)
import functools

import jax
import jax.numpy as jnp
from jax import lax
from jax.experimental import pallas as pl
from jax.experimental.pallas import tpu as pltpu
from jax.experimental.pallas import tpu_sc as plsc

CHARSET = 1000
UNIVERSAL = 1024
DIM = 128
BATCH = 4096
SEQ = 200

NC = 2
NS = 16
NW = NC * NS

ROWS_W = BATCH // NW
SPLIT = 128
REM = SEQ - SPLIT
NBUF = 3


def _matmul_body(a_ref, b_ref, o_ref):
    o_ref[...] = jnp.dot(a_ref[...], b_ref[...],
                         preferred_element_type=jnp.float32)


def _compute_table(mapping_weight, char_emb_weight):
    return pl.pallas_call(
        _matmul_body,
        out_shape=jax.ShapeDtypeStruct((CHARSET, DIM), jnp.float32),
    )(mapping_weight, char_emb_weight)


_mesh = plsc.VectorSubcoreMesh(core_axis_name="c", subcore_axis_name="s")


@functools.partial(
    pl.kernel,
    mesh=_mesh,
    out_type=jax.ShapeDtypeStruct((BATCH, SEQ, DIM), jnp.float32),
    scratch_types=[
        pltpu.VMEM((ROWS_W, SEQ), jnp.int32),
        pltpu.VMEM((NBUF, SEQ, DIM), jnp.float32),
        pltpu.VMEM_SHARED((CHARSET, DIM), jnp.float32),
        pltpu.SemaphoreType.DMA,
        pltpu.SemaphoreType.DMA,
    ],
)
def _sc_gather(table_hbm, idx_hbm, out_hbm, idx_v, rows_v, tab_sh,
               gsem, ssem):
    sid = lax.axis_index("s")
    wid = sid * NC + lax.axis_index("c")

    @pl.when(sid == 0)
    def _():
        pltpu.sync_copy(table_hbm, tab_sh)

    rbase = wid * ROWS_W
    pltpu.sync_copy(idx_hbm.at[pl.ds(rbase, ROWS_W)], idx_v)
    plsc.subcore_barrier()

    def start_gathers(r, buf):
        pltpu.async_copy(tab_sh.at[idx_v.at[r, pl.ds(0, SPLIT)]],
                         rows_v.at[buf, pl.ds(0, SPLIT)], gsem)
        pltpu.async_copy(tab_sh.at[idx_v.at[r, pl.ds(SPLIT, REM)]],
                         rows_v.at[buf, pl.ds(SPLIT, REM)], gsem)

    def wait_gathers(r, buf):
        pltpu.make_async_copy(tab_sh.at[idx_v.at[r, pl.ds(0, SPLIT)]],
                              rows_v.at[buf, pl.ds(0, SPLIT)], gsem).wait()
        pltpu.make_async_copy(tab_sh.at[idx_v.at[r, pl.ds(SPLIT, REM)]],
                              rows_v.at[buf, pl.ds(SPLIT, REM)], gsem).wait()

    def scatter_copy(r, buf):
        return pltpu.make_async_copy(rows_v.at[buf], out_hbm.at[rbase + r],
                                     ssem)

    start_gathers(0, 0)

    def body(r, carry):
        buf = lax.rem(r, NBUF)
        wait_gathers(r, buf)
        scatter_copy(r, buf).start()

        @pl.when(r + 1 < ROWS_W)
        def _():
            @pl.when(r >= 2)
            def _():
                scatter_copy(r - 2, lax.rem(r - 2, NBUF)).wait()

            start_gathers(r + 1, lax.rem(r + 1, NBUF))

        return carry

    lax.fori_loop(0, ROWS_W, body, 0)
    for t in range(ROWS_W - NBUF, ROWS_W):
        scatter_copy(t, lax.rem(t, NBUF)).wait()


def kernel(char_seq, mapping_weight, char_emb_weight):
    table = _compute_table(mapping_weight, char_emb_weight)
    return _sc_gather(table, char_seq.astype(jnp.int32))


# speedup vs baseline: 1.0305x; 1.0305x over previous
import functools

import jax
import jax.numpy as jnp
from jax import lax
from jax.experimental import pallas as pl
from jax.experimental.pallas import tpu as pltpu
from jax.experimental.pallas import tpu_sc as plsc

CHARSET = 1000
UNIVERSAL = 1024
DIM = 128
BATCH = 4096
SEQ = 200

NC = 2
NS = 16
NW = NC * NS

TOTAL = BATCH * SEQ
PER_W = TOTAL // NW
CHUNK = 128
NCHUNK = PER_W // CHUNK
NBUF = 5


def _matmul_body(a_ref, b_ref, o_ref):
    o_ref[...] = jnp.dot(a_ref[...], b_ref[...],
                         preferred_element_type=jnp.float32)


def _compute_table(mapping_weight, char_emb_weight):
    return pl.pallas_call(
        _matmul_body,
        out_shape=jax.ShapeDtypeStruct((CHARSET, DIM), jnp.float32),
    )(mapping_weight, char_emb_weight)


_mesh = plsc.VectorSubcoreMesh(core_axis_name="c", subcore_axis_name="s")


@functools.partial(
    pl.kernel,
    mesh=_mesh,
    out_type=jax.ShapeDtypeStruct((NW * NCHUNK, CHUNK, DIM), jnp.float32),
    scratch_types=[
        pltpu.VMEM((NCHUNK, CHUNK), jnp.int32),
        pltpu.VMEM((NBUF, CHUNK, DIM), jnp.float32),
        pltpu.VMEM_SHARED((CHARSET, DIM), jnp.float32),
        pltpu.SemaphoreType.DMA,
        pltpu.SemaphoreType.DMA,
    ],
)
def _sc_gather(table_hbm, idx_hbm, out_hbm, idx_v, rows_v, tab_sh,
               gsem, ssem):
    sid = lax.axis_index("s")
    wid = sid * NC + lax.axis_index("c")

    @pl.when(sid == 0)
    def _():
        pltpu.sync_copy(table_hbm, tab_sh)

    pltpu.sync_copy(idx_hbm.at[wid], idx_v)
    plsc.subcore_barrier()
    gbase = wid * NCHUNK

    def start_gather(g, buf):
        pltpu.async_copy(tab_sh.at[idx_v.at[g]], rows_v.at[buf], gsem)

    def wait_gather(g, buf):
        pltpu.make_async_copy(tab_sh.at[idx_v.at[g]], rows_v.at[buf],
                              gsem).wait()

    def scatter_copy(g, buf):
        return pltpu.make_async_copy(rows_v.at[buf], out_hbm.at[gbase + g],
                                     ssem)

    start_gather(0, 0)
    start_gather(1, 1)

    def body(g, carry):
        buf = lax.rem(g, NBUF)
        wait_gather(g, buf)
        scatter_copy(g, buf).start()

        @pl.when(g + 2 < NCHUNK)
        def _():
            @pl.when(g >= NBUF - 2)
            def _():
                scatter_copy(g - (NBUF - 2), lax.rem(g + 2, NBUF)).wait()

            start_gather(g + 2, lax.rem(g + 2, NBUF))

        return carry

    lax.fori_loop(0, NCHUNK, body, 0)
    for t in range(NCHUNK - NBUF, NCHUNK):
        scatter_copy(t, lax.rem(t, NBUF)).wait()


def kernel(char_seq, mapping_weight, char_emb_weight):
    table = _compute_table(mapping_weight, char_emb_weight)
    idx = char_seq.reshape(NW, NCHUNK, CHUNK).astype(jnp.int32)
    out = _sc_gather(table, idx)
    return out.reshape(BATCH, SEQ, DIM)


# speedup vs baseline: 1.0310x; 1.0005x over previous
import functools

import jax
import jax.numpy as jnp
from jax import lax
from jax.experimental import pallas as pl
from jax.experimental.pallas import tpu as pltpu
from jax.experimental.pallas import tpu_sc as plsc

CHARSET = 1000
UNIVERSAL = 1024
DIM = 128
BATCH = 4096
SEQ = 200

NC = 2
NS = 16
NW = NC * NS

TOTAL = BATCH * SEQ
PER_W = TOTAL // NW
CHUNK = 128
NCHUNK = PER_W // CHUNK
NBUF = 5


def _matmul_body(a_ref, b_ref, o_ref):
    o_ref[...] = jnp.dot(a_ref[...], b_ref[...],
                         preferred_element_type=jnp.float32)


def _compute_table(mapping_weight, char_emb_weight):
    return pl.pallas_call(
        _matmul_body,
        out_shape=jax.ShapeDtypeStruct((CHARSET, DIM), jnp.float32),
    )(mapping_weight, char_emb_weight)


_mesh = plsc.VectorSubcoreMesh(core_axis_name="c", subcore_axis_name="s")


@functools.partial(
    pl.kernel,
    mesh=_mesh,
    out_type=jax.ShapeDtypeStruct((NW * NCHUNK, CHUNK, DIM), jnp.float32),
    scratch_types=[
        pltpu.VMEM((NCHUNK, CHUNK), jnp.int32),
        pltpu.VMEM((NBUF, CHUNK, DIM), jnp.float32),
        pltpu.VMEM_SHARED((CHARSET, DIM), jnp.float32),
        pltpu.SemaphoreType.DMA,
        pltpu.SemaphoreType.DMA,
    ],
)
def _sc_gather(table_hbm, idx_hbm, out_hbm, idx_v, rows_v, tab_sh,
               gsem, ssem):
    sid = lax.axis_index("s")
    wid = sid * NC + lax.axis_index("c")

    @pl.when(sid < NS - 1)
    def _():
        pltpu.sync_copy(table_hbm.at[pl.ds(sid * 64, 64)],
                        tab_sh.at[pl.ds(sid * 64, 64)])

    @pl.when(sid == NS - 1)
    def _():
        pltpu.sync_copy(table_hbm.at[pl.ds((NS - 1) * 64, CHARSET - (NS - 1) * 64)],
                        tab_sh.at[pl.ds((NS - 1) * 64, CHARSET - (NS - 1) * 64)])

    pltpu.sync_copy(idx_hbm.at[wid], idx_v)
    plsc.subcore_barrier()
    gbase = wid * NCHUNK

    def start_gather(g, buf):
        pltpu.async_copy(tab_sh.at[idx_v.at[g]], rows_v.at[buf], gsem)

    def wait_gather(g, buf):
        pltpu.make_async_copy(tab_sh.at[idx_v.at[g]], rows_v.at[buf],
                              gsem).wait()

    def scatter_copy(g, buf):
        return pltpu.make_async_copy(rows_v.at[buf], out_hbm.at[gbase + g],
                                     ssem)

    start_gather(0, 0)
    start_gather(1, 1)

    def body(g, carry):
        buf = lax.rem(g, NBUF)
        wait_gather(g, buf)
        scatter_copy(g, buf).start()

        @pl.when(g + 2 < NCHUNK)
        def _():
            @pl.when(g >= NBUF - 2)
            def _():
                scatter_copy(g - (NBUF - 2), lax.rem(g + 2, NBUF)).wait()

            start_gather(g + 2, lax.rem(g + 2, NBUF))

        return carry

    lax.fori_loop(0, NCHUNK, body, 0)
    for t in range(NCHUNK - NBUF, NCHUNK):
        scatter_copy(t, lax.rem(t, NBUF)).wait()


def kernel(char_seq, mapping_weight, char_emb_weight):
    table = _compute_table(mapping_weight, char_emb_weight)
    idx = char_seq.reshape(NW, NCHUNK, CHUNK).astype(jnp.int32)
    out = _sc_gather(table, idx)
    return out.reshape(BATCH, SEQ, DIM)


# speedup vs baseline: 1.0319x; 1.0008x over previous
import functools

import jax
import jax.numpy as jnp
from jax import lax
from jax.experimental import pallas as pl
from jax.experimental.pallas import tpu as pltpu
from jax.experimental.pallas import tpu_sc as plsc

CHARSET = 1000
UNIVERSAL = 1024
DIM = 128
BATCH = 4096
SEQ = 200

NC = 2
NS = 16
NW = NC * NS

TOTAL = BATCH * SEQ
PER_W = TOTAL // NW
CHUNK = 128
NCHUNK = PER_W // CHUNK
NBUF = 5


def _matmul_body(a_ref, b_ref, o_ref):
    o_ref[...] = jnp.dot(a_ref[...], b_ref[...],
                         preferred_element_type=jnp.float32)


def _compute_table(mapping_weight, char_emb_weight):
    return pl.pallas_call(
        _matmul_body,
        out_shape=jax.ShapeDtypeStruct((CHARSET, DIM), jnp.float32),
    )(mapping_weight, char_emb_weight)


_mesh = plsc.VectorSubcoreMesh(core_axis_name="c", subcore_axis_name="s")


@functools.partial(
    pl.kernel,
    mesh=_mesh,
    out_type=jax.ShapeDtypeStruct((NW * NCHUNK, CHUNK, DIM), jnp.float32),
    scratch_types=[
        pltpu.VMEM((NCHUNK, CHUNK), jnp.int32),
        pltpu.VMEM((NBUF, CHUNK, DIM), jnp.float32),
        pltpu.VMEM_SHARED((CHARSET, DIM), jnp.float32),
        pltpu.SemaphoreType.DMA,
        pltpu.SemaphoreType.DMA,
    ],
)
def _sc_gather(table_hbm, idx_hbm, out_hbm, idx_v, rows_v, tab_sh,
               gsem, ssem):
    sid = lax.axis_index("s")
    wid = sid * NC + lax.axis_index("c")

    @pl.when(sid < NS - 1)
    def _():
        pltpu.sync_copy(table_hbm.at[pl.ds(sid * 64, 64)],
                        tab_sh.at[pl.ds(sid * 64, 64)])

    @pl.when(sid == NS - 1)
    def _():
        pltpu.sync_copy(table_hbm.at[pl.ds((NS - 1) * 64, CHARSET - (NS - 1) * 64)],
                        tab_sh.at[pl.ds((NS - 1) * 64, CHARSET - (NS - 1) * 64)])

    pltpu.sync_copy(idx_hbm.at[wid], idx_v)
    plsc.subcore_barrier()
    gbase = wid * NCHUNK

    def start_gather(g, buf):
        pltpu.async_copy(tab_sh.at[idx_v.at[g]], rows_v.at[buf], gsem)

    def wait_gather(g, buf):
        pltpu.make_async_copy(tab_sh.at[idx_v.at[g]], rows_v.at[buf],
                              gsem).wait()

    def scatter_copy(g, buf):
        return pltpu.make_async_copy(rows_v.at[buf], out_hbm.at[gbase + g],
                                     ssem)

    start_gather(0, 0)
    start_gather(1, 1)
    start_gather(2, 2)

    def body(g, carry):
        buf = lax.rem(g, NBUF)
        wait_gather(g, buf)
        scatter_copy(g, buf).start()

        @pl.when(g + 3 < NCHUNK)
        def _():
            @pl.when(g >= NBUF - 3)
            def _():
                scatter_copy(g - (NBUF - 3), lax.rem(g + 3, NBUF)).wait()

            start_gather(g + 3, lax.rem(g + 3, NBUF))

        return carry

    lax.fori_loop(0, NCHUNK, body, 0)
    for t in range(NCHUNK - NBUF, NCHUNK):
        scatter_copy(t, lax.rem(t, NBUF)).wait()


def kernel(char_seq, mapping_weight, char_emb_weight):
    table = _compute_table(mapping_weight, char_emb_weight)
    idx = char_seq.reshape(NW, NCHUNK, CHUNK).astype(jnp.int32)
    out = _sc_gather(table, idx)
    return out.reshape(BATCH, SEQ, DIM)
